# Initial kernel scaffold; baseline (speedup 1.0000x reference)
import jax
import jax.numpy as jnp
from jax.experimental import pallas as pl


def kernel(x, edge_attr, edge_index, c0_Wm, c0_bm, c0_Wa, c0_ba, c1_Wm, c1_bm, c1_Wa, c1_ba, e0_W, e0_b, e1_W, e1_b, p_W1, p_b1, p_W2, p_b2):
    raise NotImplementedError("write your pallas kernel here")


# speedup vs baseline: 2.5548x; 2.5548x over previous
import functools

import jax
import jax.numpy as jnp
from jax import lax
from jax.experimental import pallas as pl
from jax.experimental.pallas import tpu as pltpu
from jax.experimental.pallas import tpu_sc as plsc

_N = 10000
_E = 320000
_D = 128
_DE = 16
_NC = 2
_NS = 16
_NT = _NC * _NS
_EPT = _E // _NT
_CH = 80
_NCH = _EPT // _CH
_NPAD = 10240
_RPT = _NPAD // _NS

_mesh = plsc.VectorSubcoreMesh(core_axis_name="c", subcore_axis_name="s")


_CROWS = _NPAD // _D


def _make_agg(with_cnt: bool):
  outs = [
      jax.ShapeDtypeStruct((_NPAD, _D), jnp.float32),
      jax.ShapeDtypeStruct((_NPAD, _D), jnp.float32),
  ]
  scratch = [
      pltpu.VMEM((_CH,), jnp.int32),
      pltpu.VMEM((_CH,), jnp.int32),
      pltpu.VMEM((_CH, _D), jnp.float32),
      pltpu.VMEM((_CH, _D), jnp.float32),
      pltpu.VMEM((_CH, _D), jnp.float32),
      pltpu.VMEM_SHARED((_NPAD, _D), jnp.float32),
      pltpu.SemaphoreType.DMA,
      pltpu.SemaphoreType.DMA,
  ]
  if with_cnt:
    outs += [
        jax.ShapeDtypeStruct((_CROWS, _D), jnp.float32),
        jax.ShapeDtypeStruct((_CROWS, _D), jnp.float32),
    ]
    scratch += [
        pltpu.VMEM((_CROWS, _D), jnp.float32),
        pltpu.VMEM((_CROWS,), jnp.int32),
        pltpu.VMEM_SHARED((_CROWS, _D), jnp.float32),
    ]

  def agg(px_hbm, ep_hbm, src_hbm, dst_hbm, out0_hbm, out1_hbm, *rest):
    if with_cnt:
      (cnt0_hbm, cnt1_hbm, sidx, didx, pxb, epb, msgb, acc, sem0, sem1,
       cloc, ciota, ccnt) = rest
    else:
      sidx, didx, pxb, epb, msgb, acc, sem0, sem1 = rest
    cid = lax.axis_index("c")
    sid = lax.axis_index("s")
    wid = cid * _NS + sid
    zero16 = jnp.zeros((16,), jnp.float32)

    def zrow(r, carry):
      for c in range(_D // 16):
        msgb[r, pl.ds(c * 16, 16)] = zero16
      return carry

    lax.fori_loop(0, _CH, zrow, 0)
    for z in range(_RPT // _CH):
      pltpu.sync_copy(msgb, acc.at[pl.ds(sid * _RPT + z * _CH, _CH)])
    if with_cnt:
      def czrow(r, carry):
        for c in range(_D // 16):
          cloc[r, pl.ds(c * 16, 16)] = zero16
        return carry

      lax.fori_loop(0, _CROWS, czrow, 0)
      lane = lax.iota(jnp.int32, 16)
      for g in range(_CROWS // 16):
        ciota[pl.ds(g * 16, 16)] = lane + g * 16
      @pl.when(sid < _CROWS // 8)
      def _():
        pltpu.sync_copy(msgb.at[pl.ds(0, 8)], ccnt.at[pl.ds(sid * 8, 8)])
    plsc.subcore_barrier()

    def chunk(i, carry):
      base = wid * _EPT + i * _CH
      pltpu.sync_copy(src_hbm.at[pl.ds(base, _CH)], sidx)
      pltpu.sync_copy(dst_hbm.at[pl.ds(base, _CH)], didx)
      g = pltpu.async_copy(px_hbm.at[sidx], pxb, sem0)
      e = pltpu.async_copy(ep_hbm.at[pl.ds(base, _CH)], epb, sem1)
      g.wait()
      e.wait()

      def row(r, c2):
        for c in range(_D // 16):
          s = pl.ds(c * 16, 16)
          msgb[r, s] = jnp.maximum(pxb[r, s] + epb[r, s], 0.0)
        return c2

      lax.fori_loop(0, _CH, row, 0)
      if with_cnt:
        ones16 = jnp.ones((16,), jnp.float32)

        def crow(g, c2):
          d = didx[pl.ds(g * 16, 16)]
          plsc.addupdate_scatter(
              cloc, [jnp.right_shift(d, 7), jnp.bitwise_and(d, _D - 1)],
              ones16)
          return c2

        lax.fori_loop(0, _CH // 16, crow, 0)
      pltpu.sync_copy(msgb, acc.at[didx], add=True)
      return carry

    lax.fori_loop(0, _NCH, chunk, 0)
    if with_cnt:
      pltpu.sync_copy(cloc, ccnt.at[ciota], add=True)
    plsc.subcore_barrier()
    rs = pl.ds(sid * _RPT, _RPT)
    cs = pl.ds(sid * 8, 8)

    @pl.when(cid == 0)
    def _():
      pltpu.sync_copy(acc.at[rs], out0_hbm.at[rs])
      if with_cnt:
        @pl.when(sid < _CROWS // 8)
        def _():
          pltpu.sync_copy(ccnt.at[cs], cnt0_hbm.at[cs])

    @pl.when(cid == 1)
    def _():
      pltpu.sync_copy(acc.at[rs], out1_hbm.at[rs])
      if with_cnt:
        @pl.when(sid < _CROWS // 8)
        def _():
          pltpu.sync_copy(ccnt.at[cs], cnt1_hbm.at[cs])

  return functools.partial(
      pl.kernel, out_type=outs, mesh=_mesh, scratch_types=scratch,
      compiler_params=pltpu.CompilerParams(needs_layout_passes=False))(agg)


_agg_cnt = _make_agg(True)
_agg_plain = _make_agg(False)


@functools.partial(
    pl.kernel,
    out_type=jax.ShapeDtypeStruct((_E, _DE), jnp.float32),
    mesh=_mesh,
    scratch_types=[
        pltpu.VMEM((_CH,), jnp.int32),
        pltpu.VMEM((_CH,), jnp.int32),
        pltpu.VMEM((_CH, _DE), jnp.float32),
        pltpu.VMEM((_CH, _DE), jnp.float32),
        pltpu.VMEM((_CH, _DE), jnp.float32),
        pltpu.VMEM((_CH, _DE), jnp.float32),
        pltpu.SemaphoreType.DMA,
        pltpu.SemaphoreType.DMA,
        pltpu.SemaphoreType.DMA,
    ],
    compiler_params=pltpu.CompilerParams(
        needs_layout_passes=False, use_tc_tiling_on_sc=False),
)
def _edge_mlp(a_hbm, b_hbm, eap_hbm, src_hbm, dst_hbm, out_hbm,
              sidx, didx, ab, bb, eb, ob, sem0, sem1, sem2):
  cid = lax.axis_index("c")
  sid = lax.axis_index("s")
  wid = cid * _NS + sid

  def chunk(i, carry):
    base = wid * _EPT + i * _CH
    pltpu.sync_copy(src_hbm.at[pl.ds(base, _CH)], sidx)
    pltpu.sync_copy(dst_hbm.at[pl.ds(base, _CH)], didx)
    ga = pltpu.async_copy(a_hbm.at[sidx], ab, sem0)
    gb = pltpu.async_copy(b_hbm.at[didx], bb, sem1)
    ge = pltpu.async_copy(eap_hbm.at[pl.ds(base, _CH)], eb, sem2)
    ga.wait()
    gb.wait()
    ge.wait()

    def row(r, c2):
      ob[r, :] = jnp.maximum(ab[r, :] + bb[r, :] + eb[r, :], 0.0)
      return c2

    lax.fori_loop(0, _CH, row, 0)
    pltpu.sync_copy(ob, out_hbm.at[pl.ds(base, _CH)])
    return carry

  lax.fori_loop(0, _NCH, chunk, 0)


_BE = 3200
_BN = 2048


def _tc_prep_body(ea, wm, bm, we, be, ep_o, eap_o):
  e = ea[...]
  ep_o[...] = jnp.dot(e, wm[...], preferred_element_type=jnp.float32) + bm[...]
  eap_o[...] = jnp.dot(e, we[...], preferred_element_type=jnp.float32) + be[...]


def _tc_prep(ea, wm, bm, we, be):
  return pl.pallas_call(
      _tc_prep_body,
      grid=(_E // _BE,),
      in_specs=[
          pl.BlockSpec((_BE, _DE), lambda i: (i, 0)),
          pl.BlockSpec((_DE, _D), lambda i: (0, 0)),
          pl.BlockSpec((1, _D), lambda i: (0, 0)),
          pl.BlockSpec((_DE, _DE), lambda i: (0, 0)),
          pl.BlockSpec((1, _DE), lambda i: (0, 0)),
      ],
      out_specs=[
          pl.BlockSpec((_BE, _D), lambda i: (i, 0)),
          pl.BlockSpec((_BE, _DE), lambda i: (i, 0)),
      ],
      out_shape=[
          jax.ShapeDtypeStruct((_E, _D), jnp.float32),
          jax.ShapeDtypeStruct((_E, _DE), jnp.float32),
      ],
  )(ea, wm, bm, we, be)


def _tc_ep1_body(e1, wm, bm, ep_o):
  ep_o[...] = jnp.dot(e1[...], wm[...], preferred_element_type=jnp.float32) + bm[...]


def _tc_ep1(e1, wm, bm):
  return pl.pallas_call(
      _tc_ep1_body,
      grid=(_E // _BE,),
      in_specs=[
          pl.BlockSpec((_BE, _DE), lambda i: (i, 0)),
          pl.BlockSpec((_DE, _D), lambda i: (0, 0)),
          pl.BlockSpec((1, _D), lambda i: (0, 0)),
      ],
      out_specs=pl.BlockSpec((_BE, _D), lambda i: (i, 0)),
      out_shape=jax.ShapeDtypeStruct((_E, _D), jnp.float32),
  )(e1, wm, bm)


def _tc_mm_body(a, b, o):
  o[...] = jnp.dot(a[...], b[...], preferred_element_type=jnp.float32)


def _tc_mm(a, b):
  n, k = a.shape
  m = b.shape[1]
  return pl.pallas_call(
      _tc_mm_body,
      grid=(n // _BN,),
      in_specs=[
          pl.BlockSpec((_BN, k), lambda i: (i, 0)),
          pl.BlockSpec((k, m), lambda i: (0, 0)),
      ],
      out_specs=pl.BlockSpec((_BN, m), lambda i: (i, 0)),
      out_shape=jax.ShapeDtypeStruct((n, m), jnp.float32),
  )(a, b)


def _l2n(u):
  nrm = jnp.sqrt(jnp.sum(u * u, axis=1, keepdims=True))
  return u / jnp.maximum(nrm, 1e-12)


def _tc_node0_body(p0, p1, c0, c1, x, wa, wx, ba, wi, wj, wm1,
                   x1_o, px1_o, a1_o, b1_o, cnt_o):
  s = p0[...] + p1[...]
  cnt = c0[...] + c1[...]
  aggr = s / jnp.maximum(cnt, 1.0)
  u = (jnp.dot(aggr, wa[...], preferred_element_type=jnp.float32)
       + jnp.dot(x[...], wx[...], preferred_element_type=jnp.float32) + ba[...])
  x1 = _l2n(jnp.maximum(u, 0.0))
  x1_o[...] = x1
  px1_o[...] = jnp.dot(x1, wm1[...], preferred_element_type=jnp.float32)
  a1_o[...] = jnp.dot(x1, wi[...], preferred_element_type=jnp.float32)
  b1_o[...] = jnp.dot(x1, wj[...], preferred_element_type=jnp.float32)
  cnt_o[...] = cnt


def _tc_node0(p0, p1, c0, c1, xp, wa, wx, ba, wi, wj, wm1):
  return pl.pallas_call(
      _tc_node0_body,
      grid=(_NPAD // _BN,),
      in_specs=[
          pl.BlockSpec((_BN, _D), lambda i: (i, 0)),
          pl.BlockSpec((_BN, _D), lambda i: (i, 0)),
          pl.BlockSpec((_BN, 1), lambda i: (i, 0)),
          pl.BlockSpec((_BN, 1), lambda i: (i, 0)),
          pl.BlockSpec((_BN, _D), lambda i: (i, 0)),
          pl.BlockSpec((_D, _D), lambda i: (0, 0)),
          pl.BlockSpec((_D, _D), lambda i: (0, 0)),
          pl.BlockSpec((1, _D), lambda i: (0, 0)),
          pl.BlockSpec((_D, _DE), lambda i: (0, 0)),
          pl.BlockSpec((_D, _DE), lambda i: (0, 0)),
          pl.BlockSpec((_D, _D), lambda i: (0, 0)),
      ],
      out_specs=[
          pl.BlockSpec((_BN, _D), lambda i: (i, 0)),
          pl.BlockSpec((_BN, _D), lambda i: (i, 0)),
          pl.BlockSpec((_BN, _DE), lambda i: (i, 0)),
          pl.BlockSpec((_BN, _DE), lambda i: (i, 0)),
          pl.BlockSpec((_BN, 1), lambda i: (i, 0)),
      ],
      out_shape=[
          jax.ShapeDtypeStruct((_NPAD, _D), jnp.float32),
          jax.ShapeDtypeStruct((_NPAD, _D), jnp.float32),
          jax.ShapeDtypeStruct((_NPAD, _DE), jnp.float32),
          jax.ShapeDtypeStruct((_NPAD, _DE), jnp.float32),
          jax.ShapeDtypeStruct((_NPAD, 1), jnp.float32),
      ],
  )(p0, p1, c0, c1, xp, wa, wx, ba, wi, wj, wm1)


def _tc_node1_body(q0, q1, cnt, x1, wa, wx, ba, w1, b1, w2, b2, out_o):
  s = q0[...] + q1[...]
  aggr = s / jnp.maximum(cnt[...], 1.0)
  u = (jnp.dot(aggr, wa[...], preferred_element_type=jnp.float32)
       + jnp.dot(x1[...], wx[...], preferred_element_type=jnp.float32) + ba[...])
  x2 = _l2n(jnp.maximum(u, 0.0))
  h = jnp.maximum(jnp.dot(x2, w1[...], preferred_element_type=jnp.float32) + b1[...], 0.0)
  out_o[...] = jnp.dot(h, w2[...], preferred_element_type=jnp.float32) + b2[...]


def _tc_node1(q0, q1, cnt, x1, wa, wx, ba, w1, b1, w2, b2):
  return pl.pallas_call(
      _tc_node1_body,
      grid=(_NPAD // _BN,),
      in_specs=[
          pl.BlockSpec((_BN, _D), lambda i: (i, 0)),
          pl.BlockSpec((_BN, _D), lambda i: (i, 0)),
          pl.BlockSpec((_BN, 1), lambda i: (i, 0)),
          pl.BlockSpec((_BN, _D), lambda i: (i, 0)),
          pl.BlockSpec((_D, _D), lambda i: (0, 0)),
          pl.BlockSpec((_D, _D), lambda i: (0, 0)),
          pl.BlockSpec((1, _D), lambda i: (0, 0)),
          pl.BlockSpec((_D, _D), lambda i: (0, 0)),
          pl.BlockSpec((1, _D), lambda i: (0, 0)),
          pl.BlockSpec((_D, _D), lambda i: (0, 0)),
          pl.BlockSpec((1, _D), lambda i: (0, 0)),
      ],
      out_specs=pl.BlockSpec((_BN, _D), lambda i: (i, 0)),
      out_shape=jax.ShapeDtypeStruct((_NPAD, _D), jnp.float32),
  )(q0, q1, cnt, x1, wa, wx, ba, w1, b1, w2, b2)


def kernel(x, edge_attr, edge_index,
           c0_Wm, c0_bm, c0_Wa, c0_ba,
           c1_Wm, c1_bm, c1_Wa, c1_ba,
           e0_W, e0_b, e1_W, e1_b,
           p_W1, p_b1, p_W2, p_b2):
  src = edge_index[0]
  dst = edge_index[1]
  xp = jnp.pad(x, ((0, _NPAD - _N), (0, 0)))

  ep0, eap = _tc_prep(edge_attr, c0_Wm[_D:], c0_bm[None], e0_W[2 * _D:], e0_b[None])
  px0 = _tc_mm(xp, c0_Wm[:_D])
  p0, p1, ca, cb = _agg_cnt(px0, ep0, src, dst)
  x1, px1, a1, b1, cnt = _tc_node0(
      p0, p1, ca.reshape(_NPAD, 1), cb.reshape(_NPAD, 1), xp,
      c0_Wa[:_D], c0_Wa[_D:], c0_ba[None],
      e0_W[:_D], e0_W[_D:2 * _D], c1_Wm[:_D])
  e1 = _edge_mlp(a1, b1, eap, src, dst)
  ep1 = _tc_ep1(e1, c1_Wm[_D:], c1_bm[None])
  q0, q1 = _agg_plain(px1, ep1, src, dst)
  out = _tc_node1(q0, q1, cnt, x1, c1_Wa[:_D], c1_Wa[_D:], c1_ba[None],
                  p_W1, p_b1[None], p_W2, p_b2[None])
  return out[:_N]


# speedup vs baseline: 4.0121x; 1.5704x over previous
import functools

import jax
import jax.numpy as jnp
from jax import lax
from jax.experimental import pallas as pl
from jax.experimental.pallas import tpu as pltpu
from jax.experimental.pallas import tpu_sc as plsc

_N = 10000
_E = 320000
_D = 128
_DE = 16
_NC = 2
_NS = 16
_NT = _NC * _NS
_EPT = _E // _NT
_CH = 80
_NCH = _EPT // _CH
_NPAD = 10240
_RPT = _NPAD // _NS

_mesh = plsc.VectorSubcoreMesh(core_axis_name="c", subcore_axis_name="s")


_CROWS = _NPAD // _D

_sc_params = pltpu.CompilerParams(needs_layout_passes=False)


@functools.partial(
    pl.kernel,
    out_type=[
        jax.ShapeDtypeStruct((_NPAD, _D), jnp.float32),
        jax.ShapeDtypeStruct((_NPAD, _D), jnp.float32),
    ],
    mesh=_mesh,
    scratch_types=[
        pltpu.VMEM((2, _CH), jnp.int32),
        pltpu.VMEM((2, _CH), jnp.int32),
        pltpu.VMEM((2, _CH, _D), jnp.float32),
        pltpu.VMEM((2, _CH, _D), jnp.float32),
        pltpu.VMEM((2, _CH), jnp.int32),
        pltpu.VMEM_SHARED((_NPAD, _D), jnp.float32),
        pltpu.SemaphoreType.DMA,
        pltpu.SemaphoreType.DMA,
        pltpu.SemaphoreType.DMA,
        pltpu.SemaphoreType.DMA,
        pltpu.SemaphoreType.DMA,
        pltpu.SemaphoreType.DMA,
        pltpu.SemaphoreType.DMA,
        pltpu.SemaphoreType.DMA,
    ],
    compiler_params=_sc_params,
)
def _agg(px_hbm, ep_hbm, src_hbm, dst_hbm, out0_hbm, out1_hbm,
         sidx, didx, pxb, epb, sdix, acc,
         semi0, semi1, semg0, semg1, seme0, seme1, semsc0, semsc1):
  semi = (semi0, semi1)
  semg = (semg0, semg1)
  seme = (seme0, seme1)
  semsc = (semsc0, semsc1)
  cid = lax.axis_index("c")
  sid = lax.axis_index("s")
  wid = cid * _NS + sid
  zero16 = jnp.zeros((16,), jnp.float32)

  def zrow(r, carry):
    for c in range(_D // 16):
      epb[0, r, pl.ds(c * 16, 16)] = zero16
    return carry

  lax.fori_loop(0, _CH, zrow, 0)
  for z in range(_RPT // _CH):
    pltpu.sync_copy(epb.at[0], acc.at[pl.ds(sid * _RPT + z * _CH, _CH)])
  ebase = wid * _EPT

  def start_idx(i, b):
    pltpu.async_copy(src_hbm.at[pl.ds(ebase + i * _CH, _CH)],
                     sidx.at[b], semi[b])
    pltpu.async_copy(dst_hbm.at[pl.ds(ebase + i * _CH, _CH)],
                     didx.at[b], semi[b])

  def wait_idx(b):
    pltpu.make_async_copy(src_hbm.at[pl.ds(0, _CH)], sidx.at[b],
                          semi[b]).wait()
    pltpu.make_async_copy(dst_hbm.at[pl.ds(0, _CH)], didx.at[b],
                          semi[b]).wait()

  def start_rows(i, b):
    pltpu.async_copy(px_hbm.at[sidx.at[b]], pxb.at[b], semg[b])
    pltpu.async_copy(ep_hbm.at[pl.ds(ebase + i * _CH, _CH)],
                     epb.at[b], seme[b])

  def wait_rows(b):
    pltpu.make_async_copy(px_hbm.at[pl.ds(0, _CH)], pxb.at[b],
                          semg[b]).wait()
    pltpu.make_async_copy(ep_hbm.at[pl.ds(0, _CH)], epb.at[b],
                          seme[b]).wait()

  def do_chunk(i, b):
    def row(r, c2):
      for c in range(_D // 16):
        s = pl.ds(c * 16, 16)
        pxb[b, r, s] = jnp.maximum(pxb[b, r, s] + epb[b, r, s], 0.0)
      return c2

    lax.fori_loop(0, _CH, row, 0)
    for g in range(_CH // 16):
      sdix[b, pl.ds(g * 16, 16)] = didx[b, pl.ds(g * 16, 16)]
    pltpu.async_copy(pxb.at[b], acc.at[sdix.at[b]], semsc[b], add=True)

  def wait_scatter(b):
    pltpu.make_async_copy(pxb.at[b], acc.at[sdix.at[b]], semsc[b]).wait()

  start_idx(0, 0)
  wait_idx(0)
  start_rows(0, 0)
  start_idx(1, 1)
  plsc.subcore_barrier()

  def pair(g, carry):
    for b in (0, 1):
      i = g * 2 + b
      nb = 1 - b

      @pl.when(i >= 1)
      def _():
        wait_scatter(nb)
      wait_idx(nb)
      start_rows(i + 1, nb)
      wait_rows(b)
      do_chunk(i, b)

      @pl.when(i + 2 < _NCH)
      def _():
        start_idx(i + 2, b)
    return carry

  lax.fori_loop(0, (_NCH - 1) // 2, pair, 0)
  wait_rows(0)
  do_chunk(_NCH - 1, 0)
  wait_scatter(0)
  wait_scatter(1)
  plsc.subcore_barrier()
  rs = pl.ds(sid * _RPT, _RPT)

  @pl.when(cid == 0)
  def _():
    pltpu.sync_copy(acc.at[rs], out0_hbm.at[rs])

  @pl.when(cid == 1)
  def _():
    pltpu.sync_copy(acc.at[rs], out1_hbm.at[rs])


@functools.partial(
    pl.kernel,
    out_type=[
        jax.ShapeDtypeStruct((_CROWS, _D), jnp.float32),
        jax.ShapeDtypeStruct((_CROWS, _D), jnp.float32),
    ],
    mesh=_mesh,
    scratch_types=[
        pltpu.VMEM((2, _CH), jnp.int32),
        pltpu.VMEM((_CROWS, _D), jnp.float32),
        pltpu.VMEM((_CROWS,), jnp.int32),
        pltpu.VMEM_SHARED((_CROWS, _D), jnp.float32),
        pltpu.SemaphoreType.DMA,
        pltpu.SemaphoreType.DMA,
    ],
    compiler_params=_sc_params,
)
def _seg_count(dst_hbm, cnt0_hbm, cnt1_hbm, didx, cloc, ciota, ccnt,
               sem0, sem1):
  sem = (sem0, sem1)
  cid = lax.axis_index("c")
  sid = lax.axis_index("s")
  wid = cid * _NS + sid
  zero16 = jnp.zeros((16,), jnp.float32)
  ones16 = jnp.ones((16,), jnp.float32)
  lane = lax.iota(jnp.int32, 16)

  def czrow(r, carry):
    for c in range(_D // 16):
      cloc[r, pl.ds(c * 16, 16)] = zero16
    return carry

  lax.fori_loop(0, _CROWS, czrow, 0)
  for g in range(_CROWS // 16):
    ciota[pl.ds(g * 16, 16)] = lane + g * 16

  @pl.when(sid < _CROWS // 8)
  def _():
    pltpu.sync_copy(cloc.at[pl.ds(0, 8)], ccnt.at[pl.ds(sid * 8, 8)])

  ebase = wid * _EPT

  def start_idx(i, b):
    pltpu.async_copy(dst_hbm.at[pl.ds(ebase + i * _CH, _CH)],
                     didx.at[b], sem[b])

  def wait_idx(b):
    pltpu.make_async_copy(dst_hbm.at[pl.ds(0, _CH)], didx.at[b],
                          sem[b]).wait()

  def do_chunk(b):
    def crow(g, c2):
      d = didx[b, pl.ds(g * 16, 16)]
      plsc.addupdate_scatter(
          cloc, [jnp.right_shift(d, 7), jnp.bitwise_and(d, _D - 1)],
          ones16)
      return c2

    lax.fori_loop(0, _CH // 16, crow, 0)

  start_idx(0, 0)
  start_idx(1, 1)
  plsc.subcore_barrier()

  def pair(g, carry):
    for b in (0, 1):
      i = g * 2 + b
      wait_idx(b)
      do_chunk(b)

      @pl.when(i + 2 < _NCH)
      def _():
        start_idx(i + 2, b)
    return carry

  lax.fori_loop(0, (_NCH - 1) // 2, pair, 0)
  wait_idx(0)
  do_chunk(0)
  pltpu.sync_copy(cloc, ccnt.at[ciota], add=True)
  plsc.subcore_barrier()
  cs = pl.ds(sid * 8, 8)

  @pl.when(cid == 0)
  def _():
    @pl.when(sid < _CROWS // 8)
    def _():
      pltpu.sync_copy(ccnt.at[cs], cnt0_hbm.at[cs])

  @pl.when(cid == 1)
  def _():
    @pl.when(sid < _CROWS // 8)
    def _():
      pltpu.sync_copy(ccnt.at[cs], cnt1_hbm.at[cs])


@functools.partial(
    pl.kernel,
    out_type=jax.ShapeDtypeStruct((_E, _DE), jnp.float32),
    mesh=_mesh,
    scratch_types=[
        pltpu.VMEM((2, _CH), jnp.int32),
        pltpu.VMEM((2, _CH), jnp.int32),
        pltpu.VMEM((2, _CH, _DE), jnp.float32),
        pltpu.VMEM((2, _CH, _DE), jnp.float32),
        pltpu.VMEM((2, _CH, _DE), jnp.float32),
        pltpu.VMEM((2, _CH, _DE), jnp.float32),
        pltpu.SemaphoreType.DMA,
        pltpu.SemaphoreType.DMA,
        pltpu.SemaphoreType.DMA,
        pltpu.SemaphoreType.DMA,
        pltpu.SemaphoreType.DMA,
        pltpu.SemaphoreType.DMA,
    ],
    compiler_params=pltpu.CompilerParams(
        needs_layout_passes=False, use_tc_tiling_on_sc=False),
)
def _edge_mlp(a_hbm, b_hbm, eap_hbm, src_hbm, dst_hbm, out_hbm,
              sidx, didx, ab, bb, eb, ob,
              semi0, semi1, semr0, semr1, semo0, semo1):
  semi = (semi0, semi1)
  semr = (semr0, semr1)
  semo = (semo0, semo1)
  cid = lax.axis_index("c")
  sid = lax.axis_index("s")
  wid = cid * _NS + sid
  ebase = wid * _EPT

  def start_idx(i, b):
    pltpu.async_copy(src_hbm.at[pl.ds(ebase + i * _CH, _CH)],
                     sidx.at[b], semi[b])
    pltpu.async_copy(dst_hbm.at[pl.ds(ebase + i * _CH, _CH)],
                     didx.at[b], semi[b])

  def wait_idx(b):
    pltpu.make_async_copy(src_hbm.at[pl.ds(0, _CH)], sidx.at[b],
                          semi[b]).wait()
    pltpu.make_async_copy(dst_hbm.at[pl.ds(0, _CH)], didx.at[b],
                          semi[b]).wait()

  def start_rows(i, b):
    pltpu.async_copy(a_hbm.at[sidx.at[b]], ab.at[b], semr[b])
    pltpu.async_copy(b_hbm.at[didx.at[b]], bb.at[b], semr[b])
    pltpu.async_copy(eap_hbm.at[pl.ds(ebase + i * _CH, _CH)],
                     eb.at[b], semr[b])

  def wait_rows(b):
    pltpu.make_async_copy(a_hbm.at[pl.ds(0, _CH)], ab.at[b], semr[b]).wait()
    pltpu.make_async_copy(b_hbm.at[pl.ds(0, _CH)], bb.at[b], semr[b]).wait()
    pltpu.make_async_copy(eap_hbm.at[pl.ds(0, _CH)], eb.at[b],
                          semr[b]).wait()

  def do_chunk(i, b):
    def row(r, c2):
      ob[b, r, :] = jnp.maximum(ab[b, r, :] + bb[b, r, :] + eb[b, r, :], 0.0)
      return c2

    lax.fori_loop(0, _CH, row, 0)
    pltpu.async_copy(ob.at[b], out_hbm.at[pl.ds(ebase + i * _CH, _CH)],
                     semo[b])

  def wait_out(b):
    pltpu.make_async_copy(ob.at[b], out_hbm.at[pl.ds(0, _CH)],
                          semo[b]).wait()

  start_idx(0, 0)
  wait_idx(0)
  start_rows(0, 0)
  start_idx(1, 1)

  def pair(g, carry):
    for b in (0, 1):
      i = g * 2 + b
      nb = 1 - b
      wait_idx(nb)
      start_rows(i + 1, nb)
      wait_rows(b)

      @pl.when(i >= 2)
      def _():
        wait_out(b)
      do_chunk(i, b)

      @pl.when(i + 2 < _NCH)
      def _():
        start_idx(i + 2, b)
    return carry

  lax.fori_loop(0, (_NCH - 1) // 2, pair, 0)
  wait_rows(0)
  wait_out(0)
  do_chunk(_NCH - 1, 0)
  wait_out(0)
  wait_out(1)


_BE = 3200
_BN = 2048


def _tc_prep_body(ea, wm, bm, we, be, ep_o, eap_o):
  e = ea[...]
  ep_o[...] = jnp.dot(e, wm[...], preferred_element_type=jnp.float32) + bm[...]
  eap_o[...] = jnp.dot(e, we[...], preferred_element_type=jnp.float32) + be[...]


def _tc_prep(ea, wm, bm, we, be):
  return pl.pallas_call(
      _tc_prep_body,
      grid=(_E // _BE,),
      in_specs=[
          pl.BlockSpec((_BE, _DE), lambda i: (i, 0)),
          pl.BlockSpec((_DE, _D), lambda i: (0, 0)),
          pl.BlockSpec((1, _D), lambda i: (0, 0)),
          pl.BlockSpec((_DE, _DE), lambda i: (0, 0)),
          pl.BlockSpec((1, _DE), lambda i: (0, 0)),
      ],
      out_specs=[
          pl.BlockSpec((_BE, _D), lambda i: (i, 0)),
          pl.BlockSpec((_BE, _DE), lambda i: (i, 0)),
      ],
      out_shape=[
          jax.ShapeDtypeStruct((_E, _D), jnp.float32),
          jax.ShapeDtypeStruct((_E, _DE), jnp.float32),
      ],
  )(ea, wm, bm, we, be)


def _tc_ep1_body(e1, wm, bm, ep_o):
  ep_o[...] = jnp.dot(e1[...], wm[...], preferred_element_type=jnp.float32) + bm[...]


def _tc_ep1(e1, wm, bm):
  return pl.pallas_call(
      _tc_ep1_body,
      grid=(_E // _BE,),
      in_specs=[
          pl.BlockSpec((_BE, _DE), lambda i: (i, 0)),
          pl.BlockSpec((_DE, _D), lambda i: (0, 0)),
          pl.BlockSpec((1, _D), lambda i: (0, 0)),
      ],
      out_specs=pl.BlockSpec((_BE, _D), lambda i: (i, 0)),
      out_shape=jax.ShapeDtypeStruct((_E, _D), jnp.float32),
  )(e1, wm, bm)


def _tc_mm_body(a, b, o):
  o[...] = jnp.dot(a[...], b[...], preferred_element_type=jnp.float32)


def _tc_mm(a, b):
  n, k = a.shape
  m = b.shape[1]
  return pl.pallas_call(
      _tc_mm_body,
      grid=(n // _BN,),
      in_specs=[
          pl.BlockSpec((_BN, k), lambda i: (i, 0)),
          pl.BlockSpec((k, m), lambda i: (0, 0)),
      ],
      out_specs=pl.BlockSpec((_BN, m), lambda i: (i, 0)),
      out_shape=jax.ShapeDtypeStruct((n, m), jnp.float32),
  )(a, b)


def _l2n(u):
  nrm = jnp.sqrt(jnp.sum(u * u, axis=1, keepdims=True))
  return u / jnp.maximum(nrm, 1e-12)


def _tc_node0_body(p0, p1, c0, c1, x, wa, wx, ba, wi, wj, wm1,
                   x1_o, px1_o, a1_o, b1_o, cnt_o):
  s = p0[...] + p1[...]
  cnt = c0[...] + c1[...]
  aggr = s / jnp.maximum(cnt, 1.0)
  u = (jnp.dot(aggr, wa[...], preferred_element_type=jnp.float32)
       + jnp.dot(x[...], wx[...], preferred_element_type=jnp.float32) + ba[...])
  x1 = _l2n(jnp.maximum(u, 0.0))
  x1_o[...] = x1
  px1_o[...] = jnp.dot(x1, wm1[...], preferred_element_type=jnp.float32)
  a1_o[...] = jnp.dot(x1, wi[...], preferred_element_type=jnp.float32)
  b1_o[...] = jnp.dot(x1, wj[...], preferred_element_type=jnp.float32)
  cnt_o[...] = cnt


def _tc_node0(p0, p1, c0, c1, xp, wa, wx, ba, wi, wj, wm1):
  return pl.pallas_call(
      _tc_node0_body,
      grid=(_NPAD // _BN,),
      in_specs=[
          pl.BlockSpec((_BN, _D), lambda i: (i, 0)),
          pl.BlockSpec((_BN, _D), lambda i: (i, 0)),
          pl.BlockSpec((_BN, 1), lambda i: (i, 0)),
          pl.BlockSpec((_BN, 1), lambda i: (i, 0)),
          pl.BlockSpec((_BN, _D), lambda i: (i, 0)),
          pl.BlockSpec((_D, _D), lambda i: (0, 0)),
          pl.BlockSpec((_D, _D), lambda i: (0, 0)),
          pl.BlockSpec((1, _D), lambda i: (0, 0)),
          pl.BlockSpec((_D, _DE), lambda i: (0, 0)),
          pl.BlockSpec((_D, _DE), lambda i: (0, 0)),
          pl.BlockSpec((_D, _D), lambda i: (0, 0)),
      ],
      out_specs=[
          pl.BlockSpec((_BN, _D), lambda i: (i, 0)),
          pl.BlockSpec((_BN, _D), lambda i: (i, 0)),
          pl.BlockSpec((_BN, _DE), lambda i: (i, 0)),
          pl.BlockSpec((_BN, _DE), lambda i: (i, 0)),
          pl.BlockSpec((_BN, 1), lambda i: (i, 0)),
      ],
      out_shape=[
          jax.ShapeDtypeStruct((_NPAD, _D), jnp.float32),
          jax.ShapeDtypeStruct((_NPAD, _D), jnp.float32),
          jax.ShapeDtypeStruct((_NPAD, _DE), jnp.float32),
          jax.ShapeDtypeStruct((_NPAD, _DE), jnp.float32),
          jax.ShapeDtypeStruct((_NPAD, 1), jnp.float32),
      ],
  )(p0, p1, c0, c1, xp, wa, wx, ba, wi, wj, wm1)


def _tc_node1_body(q0, q1, cnt, x1, wa, wx, ba, w1, b1, w2, b2, out_o):
  s = q0[...] + q1[...]
  aggr = s / jnp.maximum(cnt[...], 1.0)
  u = (jnp.dot(aggr, wa[...], preferred_element_type=jnp.float32)
       + jnp.dot(x1[...], wx[...], preferred_element_type=jnp.float32) + ba[...])
  x2 = _l2n(jnp.maximum(u, 0.0))
  h = jnp.maximum(jnp.dot(x2, w1[...], preferred_element_type=jnp.float32) + b1[...], 0.0)
  out_o[...] = jnp.dot(h, w2[...], preferred_element_type=jnp.float32) + b2[...]


def _tc_node1(q0, q1, cnt, x1, wa, wx, ba, w1, b1, w2, b2):
  return pl.pallas_call(
      _tc_node1_body,
      grid=(_NPAD // _BN,),
      in_specs=[
          pl.BlockSpec((_BN, _D), lambda i: (i, 0)),
          pl.BlockSpec((_BN, _D), lambda i: (i, 0)),
          pl.BlockSpec((_BN, 1), lambda i: (i, 0)),
          pl.BlockSpec((_BN, _D), lambda i: (i, 0)),
          pl.BlockSpec((_D, _D), lambda i: (0, 0)),
          pl.BlockSpec((_D, _D), lambda i: (0, 0)),
          pl.BlockSpec((1, _D), lambda i: (0, 0)),
          pl.BlockSpec((_D, _D), lambda i: (0, 0)),
          pl.BlockSpec((1, _D), lambda i: (0, 0)),
          pl.BlockSpec((_D, _D), lambda i: (0, 0)),
          pl.BlockSpec((1, _D), lambda i: (0, 0)),
      ],
      out_specs=pl.BlockSpec((_BN, _D), lambda i: (i, 0)),
      out_shape=jax.ShapeDtypeStruct((_NPAD, _D), jnp.float32),
  )(q0, q1, cnt, x1, wa, wx, ba, w1, b1, w2, b2)


def kernel(x, edge_attr, edge_index,
           c0_Wm, c0_bm, c0_Wa, c0_ba,
           c1_Wm, c1_bm, c1_Wa, c1_ba,
           e0_W, e0_b, e1_W, e1_b,
           p_W1, p_b1, p_W2, p_b2):
  src = edge_index[0]
  dst = edge_index[1]
  xp = jnp.pad(x, ((0, _NPAD - _N), (0, 0)))

  ep0, eap = _tc_prep(edge_attr, c0_Wm[_D:], c0_bm[None], e0_W[2 * _D:], e0_b[None])
  px0 = _tc_mm(xp, c0_Wm[:_D])
  ca, cb = _seg_count(dst)
  p0, p1 = _agg(px0, ep0, src, dst)
  x1, px1, a1, b1, cnt = _tc_node0(
      p0, p1, ca.reshape(_NPAD, 1), cb.reshape(_NPAD, 1), xp,
      c0_Wa[:_D], c0_Wa[_D:], c0_ba[None],
      e0_W[:_D], e0_W[_D:2 * _D], c1_Wm[:_D])
  e1 = _edge_mlp(a1, b1, eap, src, dst)
  ep1 = _tc_ep1(e1, c1_Wm[_D:], c1_bm[None])
  q0, q1 = _agg(px1, ep1, src, dst)
  out = _tc_node1(q0, q1, cnt, x1, c1_Wa[:_D], c1_Wa[_D:], c1_ba[None],
                  p_W1, p_b1[None], p_W2, p_b2[None])
  return out[:_N]
